# Initial kernel scaffold; baseline (speedup 1.0000x reference)
import functools

import jax
import jax.numpy as jnp
from jax import lax
from jax.experimental import pallas as pl
from jax.experimental.pallas import tpu as pltpu
from jax.experimental.pallas import tpu_sc as plsc

N = 16384
NCAT = 26
NNUM = 13
VOCAB = 1000
C = 128
NW = 32
RPW = N // NW
RC = 4
IPC = RC * NCAT
NCH = RPW // RC
LANES = 16

_mesh = plsc.VectorSubcoreMesh(core_axis_name="c", subcore_axis_name="s")


@functools.partial(
    pl.kernel,
    mesh=_mesh,
    out_type=jax.ShapeDtypeStruct((N, NCAT + NNUM, C), jnp.float32),
    scratch_types=[
        pltpu.VMEM((NCH, IPC), jnp.int32),
        pltpu.VMEM((RPW, NNUM), jnp.float32),
        pltpu.VMEM((NNUM, C), jnp.float32),
        pltpu.VMEM((NNUM, C), jnp.float32),
        pltpu.VMEM((RC, NCAT, C), jnp.float32),
        pltpu.VMEM((RC, NNUM, C), jnp.float32),
        pltpu.SemaphoreType.DMA,
    ],
)
def _encode(table_hbm, idx_hbm, fnum_hbm, w_hbm, b_hbm, out_hbm,
            idx_v, fnum_v, w_v, b_v, gbuf, nbuf, sem):
    wid = lax.axis_index("s") * 2 + lax.axis_index("c")
    pltpu.sync_copy(idx_hbm.at[wid], idx_v)
    pltpu.sync_copy(fnum_hbm.at[wid], fnum_v)
    pltpu.sync_copy(w_hbm, w_v)
    pltpu.sync_copy(b_hbm, b_v)
    base = wid * RPW

    def chunk(c, carry):
        n0 = base + c * RC
        pltpu.async_copy(table_hbm.at[idx_v.at[c]], gbuf, sem).wait()
        for r in range(RC):
            row = c * RC + r
            for j in range(NNUM):
                v16 = jnp.full((LANES,), fnum_v[row, j], dtype=jnp.float32)
                for k in range(C // LANES):
                    s = pl.ds(k * LANES, LANES)
                    nbuf[r, j, s] = v16 * w_v[j, s] + b_v[j, s]
        pltpu.sync_copy(gbuf, out_hbm.at[pl.ds(n0, RC), pl.ds(0, NCAT)])
        pltpu.sync_copy(nbuf, out_hbm.at[pl.ds(n0, RC), pl.ds(NCAT, NNUM)])
        return carry

    lax.fori_loop(0, NCH, chunk, 0)


def kernel(feat_cat, feat_num, emb_tables, lin_weight, lin_bias, num_mean, num_std):
    table = emb_tables.reshape(NCAT * VOCAB, C)
    offs = jnp.arange(NCAT, dtype=jnp.int32) * VOCAB
    idx = (feat_cat.astype(jnp.int32) + offs[None, :]).reshape(NW, NCH, IPC)
    fnum = feat_num.reshape(NW, RPW, NNUM)
    inv = 1.0 / num_std
    w_eff = lin_weight * inv[:, None]
    b_eff = lin_bias - (num_mean * inv)[:, None] * lin_weight
    return _encode(table, idx, fnum, w_eff, b_eff)


# speedup vs baseline: 2.7127x; 2.7127x over previous
import functools

import jax
import jax.numpy as jnp
from jax import lax
from jax.experimental import pallas as pl
from jax.experimental.pallas import tpu as pltpu
from jax.experimental.pallas import tpu_sc as plsc

N = 16384
NCAT = 26
NNUM = 13
VOCAB = 1000
C = 128
NW = 32
RPW = N // NW
RC = 16
GPC = 4
IPG = RC * NCAT // GPC
NCH = RPW // RC
LANES = 16

_mesh = plsc.VectorSubcoreMesh(core_axis_name="c", subcore_axis_name="s")


@functools.partial(
    pl.kernel,
    mesh=_mesh,
    out_type=jax.ShapeDtypeStruct((N, NCAT + NNUM, C), jnp.float32),
    compiler_params=pltpu.CompilerParams(use_tc_tiling_on_sc=False),
    scratch_types=[
        pltpu.VMEM((NCH * GPC, IPG), jnp.int32),
        pltpu.VMEM((NNUM, RPW), jnp.float32),
        pltpu.VMEM((NNUM, C), jnp.float32),
        pltpu.VMEM((NNUM, C), jnp.float32),
        pltpu.VMEM((RC * NCAT, C), jnp.float32),
        pltpu.VMEM((RC, NNUM, C), jnp.float32),
        pltpu.SemaphoreType.DMA,
    ],
)
def _encode(table_hbm, idx_hbm, fnum_hbm, w_hbm, b_hbm, out_hbm,
            idx_v, fnum_v, w_v, b_v, gbuf, nbuf, sem):
    wid = lax.axis_index("s") * 2 + lax.axis_index("c")
    pltpu.sync_copy(idx_hbm.at[wid], idx_v)
    pltpu.sync_copy(fnum_hbm.at[wid], fnum_v)
    pltpu.sync_copy(w_hbm, w_v)
    pltpu.sync_copy(b_hbm, b_v)
    base = wid * RPW

    def chunk(c, carry):
        n0 = base + c * RC
        cps = [
            pltpu.async_copy(
                table_hbm.at[idx_v.at[c * GPC + q]],
                gbuf.at[pl.ds(q * IPG, IPG)],
                sem,
            )
            for q in range(GPC)
        ]

        def jbody(j, carry2):
            v16 = fnum_v[j, pl.ds(c * RC, RC)]
            for r in range(RC):
                vb = jnp.full((LANES,), v16[r], dtype=jnp.float32)
                for k in range(C // LANES):
                    s = pl.ds(k * LANES, LANES)
                    nbuf[r, j, s] = vb * w_v[j, s] + b_v[j, s]
            return carry2

        lax.fori_loop(0, NNUM, jbody, 0)
        for cp in cps:
            cp.wait()
        for r in range(RC):
            pltpu.sync_copy(gbuf.at[pl.ds(r * NCAT, NCAT)],
                            out_hbm.at[n0 + r, pl.ds(0, NCAT)])
        pltpu.sync_copy(nbuf, out_hbm.at[pl.ds(n0, RC), pl.ds(NCAT, NNUM)])
        return carry

    lax.fori_loop(0, NCH, chunk, 0)


def kernel(feat_cat, feat_num, emb_tables, lin_weight, lin_bias, num_mean, num_std):
    table = emb_tables.reshape(NCAT * VOCAB, C)
    offs = jnp.arange(NCAT, dtype=jnp.int32) * VOCAB
    idx = (feat_cat.astype(jnp.int32) + offs[None, :]).reshape(NW, NCH * GPC, IPG)
    fnum = feat_num.reshape(NW, RPW, NNUM).transpose(0, 2, 1)
    inv = 1.0 / num_std
    w_eff = lin_weight * inv[:, None]
    b_eff = lin_bias - (num_mean * inv)[:, None] * lin_weight
    return _encode(table, idx, fnum, w_eff, b_eff)
